# Initial kernel scaffold; baseline (speedup 1.0000x reference)
import jax
import jax.numpy as jnp
from jax.experimental import pallas as pl


def kernel(x, edge_index, W1, b1, W2, b2, W3, b3):
    raise NotImplementedError("write your pallas kernel here")


# speedup vs baseline: 20.8442x; 20.8442x over previous
import functools

import jax
import jax.numpy as jnp
from jax import lax
from jax.experimental import pallas as pl
from jax.experimental.pallas import tpu as pltpu
from jax.experimental.pallas import tpu_sc as plsc

N = 10000
NPAD = 10240
E = 320000
NC = 2
NS = 16
NW = NC * NS
EPW = E // NW
CE = 80
CH = EPW // CE
ZROWS = 128
SLICE = NPAD // NS

R = 400
G = N // R

_MESH = plsc.VectorSubcoreMesh(core_axis_name="c", subcore_axis_name="s")
_SC_PARAMS = pltpu.CompilerParams(use_tc_tiling_on_sc=False)


def _sc_degree(dst3):
    @functools.partial(
        pl.kernel,
        mesh=_MESH,
        out_type=jax.ShapeDtypeStruct((NC, NPAD), jnp.float32),
        compiler_params=_SC_PARAMS,
        scratch_types=[
            pltpu.VMEM((CH, CE), jnp.int32),
            pltpu.VMEM((CE,), jnp.float32),
            pltpu.VMEM((SLICE,), jnp.float32),
            pltpu.VMEM_SHARED((NPAD,), jnp.float32),
        ],
    )
    def k(dst_hbm, out_hbm, dst_v, ones_v, zbuf_v, acc):
        c = lax.axis_index("c")
        s = lax.axis_index("s")
        wid = c * NS + s
        for i in range(CE // 16):
            ones_v[pl.ds(i * 16, 16)] = jnp.ones((16,), jnp.float32)
        for i in range(SLICE // 16):
            zbuf_v[pl.ds(i * 16, 16)] = jnp.zeros((16,), jnp.float32)
        pltpu.sync_copy(zbuf_v, acc.at[pl.ds(s * SLICE, SLICE)])
        plsc.subcore_barrier()
        pltpu.sync_copy(dst_hbm.at[wid], dst_v)

        def body(g, carry):
            pltpu.sync_copy(ones_v, acc.at[dst_v.at[g]], add=True)
            return carry

        lax.fori_loop(0, CH, body, 0)
        plsc.subcore_barrier()
        pltpu.sync_copy(acc.at[pl.ds(s * SLICE, SLICE)],
                        out_hbm.at[c, pl.ds(s * SLICE, SLICE)])

    return k(dst3)


def _sc_agg(src3, dst3, xws, C):
    @functools.partial(
        pl.kernel,
        mesh=_MESH,
        out_type=jax.ShapeDtypeStruct((NC, NPAD, C), jnp.float32),
        compiler_params=_SC_PARAMS,
        scratch_types=[
            pltpu.VMEM((CH, CE), jnp.int32),
            pltpu.VMEM((CH, CE), jnp.int32),
            pltpu.VMEM((CE, C), jnp.float32),
            pltpu.VMEM((ZROWS, C), jnp.float32),
            pltpu.VMEM_SHARED((NPAD, C), jnp.float32),
            pltpu.SemaphoreType.DMA,
        ],
    )
    def k(src_hbm, dst_hbm, xws_hbm, out_hbm, src_v, dst_v, msg_v, zbuf_v,
          acc, sem):
        c = lax.axis_index("c")
        s = lax.axis_index("s")
        wid = c * NS + s

        def zrow(i, carry):
            for j in range(C // 16):
                zbuf_v[i, pl.ds(j * 16, 16)] = jnp.zeros((16,), jnp.float32)
            return carry

        lax.fori_loop(0, ZROWS, zrow, 0)
        for r in range(SLICE // ZROWS):
            pltpu.sync_copy(zbuf_v, acc.at[pl.ds(s * SLICE + r * ZROWS, ZROWS)])
        plsc.subcore_barrier()
        pltpu.sync_copy(src_hbm.at[wid], src_v)
        pltpu.sync_copy(dst_hbm.at[wid], dst_v)

        def body(g, carry):
            pltpu.async_copy(xws_hbm.at[src_v.at[g]], msg_v, sem).wait()
            pltpu.sync_copy(msg_v, acc.at[dst_v.at[g]], add=True)
            return carry

        lax.fori_loop(0, CH, body, 0)
        plsc.subcore_barrier()
        for r in range(SLICE // ZROWS):
            base = s * SLICE + r * ZROWS
            pltpu.sync_copy(acc.at[pl.ds(base, ZROWS)],
                            out_hbm.at[c, pl.ds(base, ZROWS)])

    return k(src3, dst3, xws)


def _tc1(x, degT, W1):
    def body(x_ref, degT_ref, w_ref, xws_ref, dis_ref):
        deg = degT_ref[:, 0:1] + degT_ref[:, 1:2] + 1.0
        dis = lax.rsqrt(deg)
        xw = jnp.dot(x_ref[...], w_ref[...], preferred_element_type=jnp.float32)
        xws_ref[...] = xw * dis
        dis_ref[...] = dis

    return pl.pallas_call(
        body,
        grid=(G,),
        in_specs=[
            pl.BlockSpec((R, 128), lambda i: (i, 0)),
            pl.BlockSpec((R, 2), lambda i: (i, 0)),
            pl.BlockSpec((128, 64), lambda i: (0, 0)),
        ],
        out_specs=[
            pl.BlockSpec((R, 64), lambda i: (i, 0)),
            pl.BlockSpec((R, 1), lambda i: (i, 0)),
        ],
        out_shape=[
            jax.ShapeDtypeStruct((N, 64), jnp.float32),
            jax.ShapeDtypeStruct((N, 1), jnp.float32),
        ],
    )(x, degT, W1)


def _tc_mid(parts, xws, dis, b, W, C_in, C_out):
    def body(p_ref, xws_ref, dis_ref, b_ref, w_ref, out_ref):
        agg = p_ref[0] + p_ref[1] + xws_ref[...]
        h = jnp.maximum(dis_ref[...] * agg + b_ref[...], 0.0)
        hw = jnp.dot(h, w_ref[...], preferred_element_type=jnp.float32)
        out_ref[...] = hw * dis_ref[...]

    return pl.pallas_call(
        body,
        grid=(G,),
        in_specs=[
            pl.BlockSpec((NC, R, C_in), lambda i: (0, i, 0)),
            pl.BlockSpec((R, C_in), lambda i: (i, 0)),
            pl.BlockSpec((R, 1), lambda i: (i, 0)),
            pl.BlockSpec((1, C_in), lambda i: (0, 0)),
            pl.BlockSpec((C_in, C_out), lambda i: (0, 0)),
        ],
        out_specs=pl.BlockSpec((R, C_out), lambda i: (i, 0)),
        out_shape=jax.ShapeDtypeStruct((N, C_out), jnp.float32),
    )(parts, xws, dis, b, W)


def _tc_scale(parts, xws, dis, b, C):
    def body(p_ref, xws_ref, dis_ref, b_ref, out_ref):
        agg = p_ref[0] + p_ref[1] + xws_ref[...]
        h = jnp.maximum(dis_ref[...] * agg + b_ref[...], 0.0)
        out_ref[...] = h * dis_ref[...]

    return pl.pallas_call(
        body,
        grid=(G,),
        in_specs=[
            pl.BlockSpec((NC, R, C), lambda i: (0, i, 0)),
            pl.BlockSpec((R, C), lambda i: (i, 0)),
            pl.BlockSpec((R, 1), lambda i: (i, 0)),
            pl.BlockSpec((1, C), lambda i: (0, 0)),
        ],
        out_specs=pl.BlockSpec((R, C), lambda i: (i, 0)),
        out_shape=jax.ShapeDtypeStruct((N, C), jnp.float32),
    )(parts, xws, dis, b)


def _tc_final(parts, xws, dis, W3, b3, C_in):
    def body(p_ref, xws_ref, dis_ref, w_ref, b_ref, out_ref):
        agg = p_ref[0] + p_ref[1] + xws_ref[...]
        t = dis_ref[...] * agg
        tw = jnp.dot(t, w_ref[...], preferred_element_type=jnp.float32)
        out_ref[...] = jnp.maximum(tw + b_ref[...], 0.0)

    return pl.pallas_call(
        body,
        grid=(G,),
        in_specs=[
            pl.BlockSpec((NC, R, C_in), lambda i: (0, i, 0)),
            pl.BlockSpec((R, C_in), lambda i: (i, 0)),
            pl.BlockSpec((R, 1), lambda i: (i, 0)),
            pl.BlockSpec((C_in, 2), lambda i: (0, 0)),
            pl.BlockSpec((1, 2), lambda i: (0, 0)),
        ],
        out_specs=pl.BlockSpec((R, 2), lambda i: (i, 0)),
        out_shape=jax.ShapeDtypeStruct((N, 2), jnp.float32),
    )(parts, xws, dis, W3, b3)


def kernel(x, edge_index, W1, b1, W2, b2, W3, b3):
    src3 = edge_index[0].astype(jnp.int32).reshape(NW, CH, CE)
    dst3 = edge_index[1].astype(jnp.int32).reshape(NW, CH, CE)

    degp = _sc_degree(dst3)
    degT = jnp.transpose(degp[:, :N])

    xws1, dis = _tc1(x, degT, W1)
    p1 = _sc_agg(src3, dst3, xws1, 64)
    xws2 = _tc_mid(p1[:, :N, :], xws1, dis, b1.reshape(1, -1), W2, 64, 32)
    p2 = _sc_agg(src3, dst3, xws2, 32)
    xws3 = _tc_scale(p2[:, :N, :], xws2, dis, b2.reshape(1, -1), 32)
    p3 = _sc_agg(src3, dst3, xws3, 32)
    return _tc_final(p3[:, :N, :], xws3, dis, W3, b3.reshape(1, -1), 32)


# speedup vs baseline: 33.5271x; 1.6085x over previous
import functools

import jax
import jax.numpy as jnp
from jax import lax
from jax.experimental import pallas as pl
from jax.experimental.pallas import tpu as pltpu
from jax.experimental.pallas import tpu_sc as plsc

N = 10000
NPAD = 10240
E = 320000
NC = 2
NS = 16
NW = NC * NS
EPW = E // NW
CE = 100
CH = EPW // CE
K = 5
SLICE = NPAD // NS

R = 400
G = N // R

_MESH = plsc.VectorSubcoreMesh(core_axis_name="c", subcore_axis_name="s")
_SC_PARAMS = pltpu.CompilerParams(use_tc_tiling_on_sc=False)


def _sc_degree(dst3):
    @functools.partial(
        pl.kernel,
        mesh=_MESH,
        out_type=jax.ShapeDtypeStruct((NC, NPAD), jnp.float32),
        compiler_params=_SC_PARAMS,
        scratch_types=[
            pltpu.VMEM((CH, CE), jnp.int32),
            pltpu.VMEM((112,), jnp.float32),
            pltpu.VMEM((SLICE,), jnp.float32),
            pltpu.VMEM_SHARED((NPAD,), jnp.float32),
        ],
    )
    def k(dst_hbm, out_hbm, dst_v, ones_v, zbuf_v, acc):
        c = lax.axis_index("c")
        s = lax.axis_index("s")
        wid = c * NS + s
        for i in range(112 // 16):
            ones_v[pl.ds(i * 16, 16)] = jnp.ones((16,), jnp.float32)
        for i in range(SLICE // 16):
            zbuf_v[pl.ds(i * 16, 16)] = jnp.zeros((16,), jnp.float32)
        pltpu.sync_copy(zbuf_v, acc.at[pl.ds(s * SLICE, SLICE)])
        plsc.subcore_barrier()
        pltpu.sync_copy(dst_hbm.at[wid], dst_v)

        def body(g, carry):
            pltpu.sync_copy(ones_v.at[pl.ds(0, CE)], acc.at[dst_v.at[g]],
                            add=True)
            return carry

        lax.fori_loop(0, CH, body, 0)
        plsc.subcore_barrier()
        pltpu.sync_copy(acc.at[pl.ds(s * SLICE, SLICE)],
                        out_hbm.at[c, pl.ds(s * SLICE, SLICE)])

    return k(dst3)


def _sc_agg(src3, dst3, xws, C):
    @functools.partial(
        pl.kernel,
        mesh=_MESH,
        out_type=jax.ShapeDtypeStruct((NC, NPAD, C), jnp.float32),
        compiler_params=_SC_PARAMS,
        scratch_types=[
            pltpu.VMEM((CH, CE), jnp.int32),
            pltpu.VMEM((CH, CE), jnp.int32),
            pltpu.VMEM((K, CE, C), jnp.float32),
            pltpu.VMEM((128, C), jnp.float32),
            pltpu.VMEM_SHARED((NPAD, C), jnp.float32),
            pltpu.SemaphoreType.DMA((K,)),
            pltpu.SemaphoreType.DMA((K,)),
        ],
    )
    def k(src_hbm, dst_hbm, xws_hbm, out_hbm, src_v, dst_v, msg_v, zbuf_v,
          acc, gsem, ssem):
        c = lax.axis_index("c")
        s = lax.axis_index("s")
        wid = c * NS + s

        def zrow(i, carry):
            for j in range(C // 16):
                zbuf_v[i, pl.ds(j * 16, 16)] = jnp.zeros((16,), jnp.float32)
            return carry

        lax.fori_loop(0, 128, zrow, 0)
        for r in range(SLICE // 128):
            pltpu.sync_copy(zbuf_v, acc.at[pl.ds(s * SLICE + r * 128, 128)])
        plsc.subcore_barrier()
        pltpu.sync_copy(src_hbm.at[wid], src_v)
        pltpu.sync_copy(dst_hbm.at[wid], dst_v)

        def body(i, carry):
            g0 = i * K
            gathers = [
                pltpu.async_copy(xws_hbm.at[src_v.at[g0 + kk]],
                                 msg_v.at[kk], gsem.at[kk])
                for kk in range(K)
            ]
            scatters = []
            for kk in range(K):
                gathers[kk].wait()
                scatters.append(
                    pltpu.async_copy(msg_v.at[kk], acc.at[dst_v.at[g0 + kk]],
                                     ssem.at[kk], add=True))
            for kk in range(K):
                scatters[kk].wait()
            return carry

        lax.fori_loop(0, CH // K, body, 0)
        plsc.subcore_barrier()
        pltpu.sync_copy(acc.at[pl.ds(s * SLICE, SLICE)],
                        out_hbm.at[c, pl.ds(s * SLICE, SLICE)])

    return k(src3, dst3, xws)


def _tc1(x, degT, W1):
    def body(x_ref, degT_ref, w_ref, xws_ref, dis_ref):
        deg = degT_ref[:, 0:1] + degT_ref[:, 1:2] + 1.0
        dis = lax.rsqrt(deg)
        xw = jnp.dot(x_ref[...], w_ref[...], preferred_element_type=jnp.float32)
        xws_ref[...] = xw * dis
        dis_ref[...] = dis

    return pl.pallas_call(
        body,
        grid=(G,),
        in_specs=[
            pl.BlockSpec((R, 128), lambda i: (i, 0)),
            pl.BlockSpec((R, 2), lambda i: (i, 0)),
            pl.BlockSpec((128, 64), lambda i: (0, 0)),
        ],
        out_specs=[
            pl.BlockSpec((R, 64), lambda i: (i, 0)),
            pl.BlockSpec((R, 1), lambda i: (i, 0)),
        ],
        out_shape=[
            jax.ShapeDtypeStruct((N, 64), jnp.float32),
            jax.ShapeDtypeStruct((N, 1), jnp.float32),
        ],
    )(x, degT, W1)


def _tc_mid(parts, xws, dis, b, W, C_in, C_out):
    def body(p_ref, xws_ref, dis_ref, b_ref, w_ref, out_ref):
        agg = p_ref[0] + p_ref[1] + xws_ref[...]
        h = jnp.maximum(dis_ref[...] * agg + b_ref[...], 0.0)
        hw = jnp.dot(h, w_ref[...], preferred_element_type=jnp.float32)
        out_ref[...] = hw * dis_ref[...]

    return pl.pallas_call(
        body,
        grid=(G,),
        in_specs=[
            pl.BlockSpec((NC, R, C_in), lambda i: (0, i, 0)),
            pl.BlockSpec((R, C_in), lambda i: (i, 0)),
            pl.BlockSpec((R, 1), lambda i: (i, 0)),
            pl.BlockSpec((1, C_in), lambda i: (0, 0)),
            pl.BlockSpec((C_in, C_out), lambda i: (0, 0)),
        ],
        out_specs=pl.BlockSpec((R, C_out), lambda i: (i, 0)),
        out_shape=jax.ShapeDtypeStruct((N, C_out), jnp.float32),
    )(parts, xws, dis, b, W)


def _tc_scale(parts, xws, dis, b, C):
    def body(p_ref, xws_ref, dis_ref, b_ref, out_ref):
        agg = p_ref[0] + p_ref[1] + xws_ref[...]
        h = jnp.maximum(dis_ref[...] * agg + b_ref[...], 0.0)
        out_ref[...] = h * dis_ref[...]

    return pl.pallas_call(
        body,
        grid=(G,),
        in_specs=[
            pl.BlockSpec((NC, R, C), lambda i: (0, i, 0)),
            pl.BlockSpec((R, C), lambda i: (i, 0)),
            pl.BlockSpec((R, 1), lambda i: (i, 0)),
            pl.BlockSpec((1, C), lambda i: (0, 0)),
        ],
        out_specs=pl.BlockSpec((R, C), lambda i: (i, 0)),
        out_shape=jax.ShapeDtypeStruct((N, C), jnp.float32),
    )(parts, xws, dis, b)


def _tc_final(parts, xws, dis, W3, b3, C_in):
    def body(p_ref, xws_ref, dis_ref, w_ref, b_ref, out_ref):
        agg = p_ref[0] + p_ref[1] + xws_ref[...]
        t = dis_ref[...] * agg
        tw = jnp.dot(t, w_ref[...], preferred_element_type=jnp.float32)
        out_ref[...] = jnp.maximum(tw + b_ref[...], 0.0)

    return pl.pallas_call(
        body,
        grid=(G,),
        in_specs=[
            pl.BlockSpec((NC, R, C_in), lambda i: (0, i, 0)),
            pl.BlockSpec((R, C_in), lambda i: (i, 0)),
            pl.BlockSpec((R, 1), lambda i: (i, 0)),
            pl.BlockSpec((C_in, 2), lambda i: (0, 0)),
            pl.BlockSpec((1, 2), lambda i: (0, 0)),
        ],
        out_specs=pl.BlockSpec((R, 2), lambda i: (i, 0)),
        out_shape=jax.ShapeDtypeStruct((N, 2), jnp.float32),
    )(parts, xws, dis, W3, b3)


def kernel(x, edge_index, W1, b1, W2, b2, W3, b3):
    src3 = edge_index[0].astype(jnp.int32).reshape(NW, CH, CE)
    dst3 = edge_index[1].astype(jnp.int32).reshape(NW, CH, CE)

    degp = _sc_degree(dst3)
    degT = jnp.transpose(degp[:, :N])

    xws1, dis = _tc1(x, degT, W1)
    p1 = _sc_agg(src3, dst3, xws1, 64)
    xws2 = _tc_mid(p1[:, :N, :], xws1, dis, b1.reshape(1, -1), W2, 64, 32)
    p2 = _sc_agg(src3, dst3, xws2, 32)
    xws3 = _tc_scale(p2[:, :N, :], xws2, dis, b2.reshape(1, -1), 32)
    p3 = _sc_agg(src3, dst3, xws3, 32)
    return _tc_final(p3[:, :N, :], xws3, dis, W3, b3.reshape(1, -1), 32)


# speedup vs baseline: 39.8154x; 1.1876x over previous
import functools

import jax
import jax.numpy as jnp
from jax import lax
from jax.experimental import pallas as pl
from jax.experimental.pallas import tpu as pltpu
from jax.experimental.pallas import tpu_sc as plsc

N = 10000
NPAD = 10240
E = 320000
NC = 2
NS = 16
NW = NC * NS
EPW = E // NW
CE = 100
CH = EPW // CE
K = 5
SLICE = NPAD // NS

R = 2000
G = N // R

_MESH = plsc.VectorSubcoreMesh(core_axis_name="c", subcore_axis_name="s")
_SC_PARAMS = pltpu.CompilerParams(use_tc_tiling_on_sc=False)


def _sc_degree(dst3):
    @functools.partial(
        pl.kernel,
        mesh=_MESH,
        out_type=jax.ShapeDtypeStruct((NC, NPAD), jnp.float32),
        compiler_params=_SC_PARAMS,
        scratch_types=[
            pltpu.VMEM((CH, CE), jnp.int32),
            pltpu.VMEM((112,), jnp.float32),
            pltpu.VMEM((SLICE,), jnp.float32),
            pltpu.VMEM_SHARED((NPAD,), jnp.float32),
        ],
    )
    def k(dst_hbm, out_hbm, dst_v, ones_v, zbuf_v, acc):
        c = lax.axis_index("c")
        s = lax.axis_index("s")
        wid = c * NS + s
        for i in range(112 // 16):
            ones_v[pl.ds(i * 16, 16)] = jnp.ones((16,), jnp.float32)
        for i in range(SLICE // 16):
            zbuf_v[pl.ds(i * 16, 16)] = jnp.zeros((16,), jnp.float32)
        pltpu.sync_copy(zbuf_v, acc.at[pl.ds(s * SLICE, SLICE)])
        plsc.subcore_barrier()
        pltpu.sync_copy(dst_hbm.at[wid], dst_v)

        def body(g, carry):
            pltpu.sync_copy(ones_v.at[pl.ds(0, CE)], acc.at[dst_v.at[g]],
                            add=True)
            return carry

        lax.fori_loop(0, CH, body, 0)
        plsc.subcore_barrier()
        pltpu.sync_copy(acc.at[pl.ds(s * SLICE, SLICE)],
                        out_hbm.at[c, pl.ds(s * SLICE, SLICE)])

    return k(dst3)


def _sc_agg(src3, dst3, xws, C):
    @functools.partial(
        pl.kernel,
        mesh=_MESH,
        out_type=jax.ShapeDtypeStruct((NC, NPAD, C), jnp.float32),
        compiler_params=_SC_PARAMS,
        scratch_types=[
            pltpu.VMEM((CH, CE), jnp.int32),
            pltpu.VMEM((CH, CE), jnp.int32),
            pltpu.VMEM((K, CE, C), jnp.float32),
            pltpu.VMEM_SHARED((NPAD, C), jnp.float32),
            pltpu.SemaphoreType.DMA((K,)),
            pltpu.SemaphoreType.DMA((K,)),
        ],
    )
    def k(src_hbm, dst_hbm, xws_hbm, z_hbm, out_hbm, src_v, dst_v, msg_v,
          acc, gsem, ssem):
        c = lax.axis_index("c")
        s = lax.axis_index("s")
        wid = c * NS + s
        pltpu.sync_copy(z_hbm, acc.at[pl.ds(s * SLICE, SLICE)])
        plsc.subcore_barrier()
        pltpu.sync_copy(src_hbm.at[wid], src_v)
        pltpu.sync_copy(dst_hbm.at[wid], dst_v)

        def body(i, carry):
            g0 = i * K
            gathers = [
                pltpu.async_copy(xws_hbm.at[src_v.at[g0 + kk]],
                                 msg_v.at[kk], gsem.at[kk])
                for kk in range(K)
            ]
            scatters = []
            for kk in range(K):
                gathers[kk].wait()
                scatters.append(
                    pltpu.async_copy(msg_v.at[kk], acc.at[dst_v.at[g0 + kk]],
                                     ssem.at[kk], add=True))
            for kk in range(K):
                scatters[kk].wait()
            return carry

        lax.fori_loop(0, CH // K, body, 0)
        plsc.subcore_barrier()
        pltpu.sync_copy(acc.at[pl.ds(s * SLICE, SLICE)],
                        out_hbm.at[c, pl.ds(s * SLICE, SLICE)])

    return k(src3, dst3, xws, jnp.zeros((SLICE, C), jnp.float32))


def _tc1(x, degT, W1):
    def body(x_ref, degT_ref, w_ref, xws_ref, dis_ref):
        deg = degT_ref[0] + degT_ref[1] + 1.0
        dis = 1.0 / jnp.sqrt(deg)
        xw = jnp.dot(x_ref[...], w_ref[...], preferred_element_type=jnp.float32)
        xws_ref[...] = xw * dis
        dis_ref[...] = dis

    return pl.pallas_call(
        body,
        grid=(G,),
        in_specs=[
            pl.BlockSpec((R, 128), lambda i: (i, 0)),
            pl.BlockSpec((NC, R, 1), lambda i: (0, i, 0)),
            pl.BlockSpec((128, 64), lambda i: (0, 0)),
        ],
        out_specs=[
            pl.BlockSpec((R, 64), lambda i: (i, 0)),
            pl.BlockSpec((R, 1), lambda i: (i, 0)),
        ],
        out_shape=[
            jax.ShapeDtypeStruct((N, 64), jnp.float32),
            jax.ShapeDtypeStruct((N, 1), jnp.float32),
        ],
    )(x, degT, W1)


def _tc_mid(parts, xws, dis, b, W, C_in, C_out):
    def body(p_ref, xws_ref, dis_ref, b_ref, w_ref, out_ref):
        agg = p_ref[0] + p_ref[1] + xws_ref[...]
        h = jnp.maximum(dis_ref[...] * agg + b_ref[...], 0.0)
        hw = jnp.dot(h, w_ref[...], preferred_element_type=jnp.float32)
        out_ref[...] = hw * dis_ref[...]

    return pl.pallas_call(
        body,
        grid=(G,),
        in_specs=[
            pl.BlockSpec((NC, R, C_in), lambda i: (0, i, 0)),
            pl.BlockSpec((R, C_in), lambda i: (i, 0)),
            pl.BlockSpec((R, 1), lambda i: (i, 0)),
            pl.BlockSpec((1, C_in), lambda i: (0, 0)),
            pl.BlockSpec((C_in, C_out), lambda i: (0, 0)),
        ],
        out_specs=pl.BlockSpec((R, C_out), lambda i: (i, 0)),
        out_shape=jax.ShapeDtypeStruct((N, C_out), jnp.float32),
    )(parts, xws, dis, b, W)


def _tc_last(parts, xws, dis, b):
    def body(p_ref, xws_ref, dis_ref, b_ref, out_ref):
        agg = (p_ref[0] + p_ref[1] + xws_ref[...])[:, :2]
        out_ref[...] = jnp.maximum(dis_ref[...] * agg + b_ref[...], 0.0)

    return pl.pallas_call(
        body,
        grid=(G,),
        in_specs=[
            pl.BlockSpec((NC, R, 16), lambda i: (0, i, 0)),
            pl.BlockSpec((R, 16), lambda i: (i, 0)),
            pl.BlockSpec((R, 1), lambda i: (i, 0)),
            pl.BlockSpec((1, 2), lambda i: (0, 0)),
        ],
        out_specs=pl.BlockSpec((R, 2), lambda i: (i, 0)),
        out_shape=jax.ShapeDtypeStruct((N, 2), jnp.float32),
    )(parts, xws, dis, b)


def kernel(x, edge_index, W1, b1, W2, b2, W3, b3):
    src3 = edge_index[0].astype(jnp.int32).reshape(NW, CH, CE)
    dst3 = edge_index[1].astype(jnp.int32).reshape(NW, CH, CE)

    degp = _sc_degree(dst3).reshape(NC, NPAD, 1)

    xws1, dis = _tc1(x, degp, W1)
    p1 = _sc_agg(src3, dst3, xws1, 64)
    xws2 = _tc_mid(p1, xws1, dis, b1.reshape(1, -1), W2, 64, 32)
    p2 = _sc_agg(src3, dst3, xws2, 32)
    W3p = jnp.concatenate([W3, jnp.zeros((W3.shape[0], 14), W3.dtype)], axis=1)
    xws3 = _tc_mid(p2, xws2, dis, b2.reshape(1, -1), W3p, 32, 16)
    p3 = _sc_agg(src3, dst3, xws3, 16)
    return _tc_last(p3, xws3, dis, b3.reshape(1, -1))


# speedup vs baseline: 44.9340x; 1.1286x over previous
import functools

import jax
import jax.numpy as jnp
from jax import lax
from jax.experimental import pallas as pl
from jax.experimental.pallas import tpu as pltpu
from jax.experimental.pallas import tpu_sc as plsc

N = 10000
NPAD = 10240
E = 320000
NC = 2
NS = 16
NW = NC * NS
EPW = E // NW
CE = 100
CH = EPW // CE
K = 10
SLICE = NPAD // NS

R = 2000
G = N // R

_MESH = plsc.VectorSubcoreMesh(core_axis_name="c", subcore_axis_name="s")
_SC_PARAMS = pltpu.CompilerParams(use_tc_tiling_on_sc=False)


def _sc_degree(dst3):
    @functools.partial(
        pl.kernel,
        mesh=_MESH,
        out_type=jax.ShapeDtypeStruct((NC, NPAD), jnp.float32),
        compiler_params=_SC_PARAMS,
        scratch_types=[
            pltpu.VMEM((CH, CE), jnp.int32),
            pltpu.VMEM((112,), jnp.float32),
            pltpu.VMEM((SLICE,), jnp.float32),
            pltpu.VMEM_SHARED((NPAD,), jnp.float32),
        ],
    )
    def k(dst_hbm, out_hbm, dst_v, ones_v, zbuf_v, acc):
        c = lax.axis_index("c")
        s = lax.axis_index("s")
        wid = c * NS + s
        for i in range(112 // 16):
            ones_v[pl.ds(i * 16, 16)] = jnp.ones((16,), jnp.float32)
        for i in range(SLICE // 16):
            zbuf_v[pl.ds(i * 16, 16)] = jnp.zeros((16,), jnp.float32)
        pltpu.sync_copy(zbuf_v, acc.at[pl.ds(s * SLICE, SLICE)])
        plsc.subcore_barrier()
        pltpu.sync_copy(dst_hbm.at[wid], dst_v)

        def body(g, carry):
            pltpu.sync_copy(ones_v.at[pl.ds(0, CE)], acc.at[dst_v.at[g]],
                            add=True)
            return carry

        lax.fori_loop(0, CH, body, 0)
        plsc.subcore_barrier()
        pltpu.sync_copy(acc.at[pl.ds(s * SLICE, SLICE)],
                        out_hbm.at[c, pl.ds(s * SLICE, SLICE)])

    return k(dst3)


def _sc_agg(src3, dst3, xws, C):
    @functools.partial(
        pl.kernel,
        mesh=_MESH,
        out_type=jax.ShapeDtypeStruct((NC, NPAD, C), jnp.float32),
        compiler_params=_SC_PARAMS,
        scratch_types=[
            pltpu.VMEM((CH, CE), jnp.int32),
            pltpu.VMEM((CH, CE), jnp.int32),
            pltpu.VMEM((K, CE, C), jnp.float32),
            pltpu.VMEM_SHARED((NPAD, C), jnp.float32),
            pltpu.SemaphoreType.DMA((K,)),
            pltpu.SemaphoreType.DMA((K,)),
        ],
    )
    def k(src_hbm, dst_hbm, xws_hbm, z_hbm, out_hbm, src_v, dst_v, msg_v,
          acc, gsem, ssem):
        c = lax.axis_index("c")
        s = lax.axis_index("s")
        wid = c * NS + s
        pltpu.sync_copy(z_hbm, acc.at[pl.ds(s * SLICE, SLICE)])
        plsc.subcore_barrier()
        pltpu.sync_copy(src_hbm.at[wid], src_v)
        pltpu.sync_copy(dst_hbm.at[wid], dst_v)

        def body(i, carry):
            g0 = i * K
            for kk in range(K):
                @pl.when(i > 0)
                def _():
                    pltpu.make_async_copy(
                        msg_v.at[kk], acc.at[dst_v.at[g0 - K + kk]],
                        ssem.at[kk]).wait()
                pltpu.async_copy(xws_hbm.at[src_v.at[g0 + kk]],
                                 msg_v.at[kk], gsem.at[kk])
            for kk in range(K):
                pltpu.make_async_copy(xws_hbm.at[src_v.at[g0 + kk]],
                                      msg_v.at[kk], gsem.at[kk]).wait()
                pltpu.async_copy(msg_v.at[kk], acc.at[dst_v.at[g0 + kk]],
                                 ssem.at[kk], add=True)
            return carry

        lax.fori_loop(0, CH // K, body, 0)
        for kk in range(K):
            pltpu.make_async_copy(
                msg_v.at[kk], acc.at[dst_v.at[CH - K + kk]],
                ssem.at[kk]).wait()
        plsc.subcore_barrier()
        pltpu.sync_copy(acc.at[pl.ds(s * SLICE, SLICE)],
                        out_hbm.at[c, pl.ds(s * SLICE, SLICE)])

    return k(src3, dst3, xws, jnp.zeros((SLICE, C), jnp.float32))


def _tc1(x, degT, W1):
    def body(x_ref, degT_ref, w_ref, xws_ref, dis_ref):
        deg = degT_ref[0] + degT_ref[1] + 1.0
        dis = 1.0 / jnp.sqrt(deg)
        xw = jnp.dot(x_ref[...], w_ref[...], preferred_element_type=jnp.float32)
        xws_ref[...] = xw * dis
        dis_ref[...] = dis

    return pl.pallas_call(
        body,
        grid=(G,),
        in_specs=[
            pl.BlockSpec((R, 128), lambda i: (i, 0)),
            pl.BlockSpec((NC, R, 1), lambda i: (0, i, 0)),
            pl.BlockSpec((128, 64), lambda i: (0, 0)),
        ],
        out_specs=[
            pl.BlockSpec((R, 64), lambda i: (i, 0)),
            pl.BlockSpec((R, 1), lambda i: (i, 0)),
        ],
        out_shape=[
            jax.ShapeDtypeStruct((N, 64), jnp.float32),
            jax.ShapeDtypeStruct((N, 1), jnp.float32),
        ],
    )(x, degT, W1)


def _tc_mid(parts, xws, dis, b, W, C_in, C_out):
    def body(p_ref, xws_ref, dis_ref, b_ref, w_ref, out_ref):
        agg = p_ref[0] + p_ref[1] + xws_ref[...]
        h = jnp.maximum(dis_ref[...] * agg + b_ref[...], 0.0)
        hw = jnp.dot(h, w_ref[...], preferred_element_type=jnp.float32)
        out_ref[...] = hw * dis_ref[...]

    return pl.pallas_call(
        body,
        grid=(G,),
        in_specs=[
            pl.BlockSpec((NC, R, C_in), lambda i: (0, i, 0)),
            pl.BlockSpec((R, C_in), lambda i: (i, 0)),
            pl.BlockSpec((R, 1), lambda i: (i, 0)),
            pl.BlockSpec((1, C_in), lambda i: (0, 0)),
            pl.BlockSpec((C_in, C_out), lambda i: (0, 0)),
        ],
        out_specs=pl.BlockSpec((R, C_out), lambda i: (i, 0)),
        out_shape=jax.ShapeDtypeStruct((N, C_out), jnp.float32),
    )(parts, xws, dis, b, W)


def _tc_last(parts, xws, dis, b):
    def body(p_ref, xws_ref, dis_ref, b_ref, out_ref):
        agg = (p_ref[0] + p_ref[1] + xws_ref[...])[:, :2]
        out_ref[...] = jnp.maximum(dis_ref[...] * agg + b_ref[...], 0.0)

    return pl.pallas_call(
        body,
        grid=(G,),
        in_specs=[
            pl.BlockSpec((NC, R, 16), lambda i: (0, i, 0)),
            pl.BlockSpec((R, 16), lambda i: (i, 0)),
            pl.BlockSpec((R, 1), lambda i: (i, 0)),
            pl.BlockSpec((1, 2), lambda i: (0, 0)),
        ],
        out_specs=pl.BlockSpec((R, 2), lambda i: (i, 0)),
        out_shape=jax.ShapeDtypeStruct((N, 2), jnp.float32),
    )(parts, xws, dis, b)


def kernel(x, edge_index, W1, b1, W2, b2, W3, b3):
    src3 = edge_index[0].astype(jnp.int32).reshape(NW, CH, CE)
    dst3 = edge_index[1].astype(jnp.int32).reshape(NW, CH, CE)

    degp = _sc_degree(dst3).reshape(NC, NPAD, 1)

    xws1, dis = _tc1(x, degp, W1)
    p1 = _sc_agg(src3, dst3, xws1, 64)
    xws2 = _tc_mid(p1, xws1, dis, b1.reshape(1, -1), W2, 64, 32)
    p2 = _sc_agg(src3, dst3, xws2, 32)
    W3p = jnp.concatenate([W3, jnp.zeros((W3.shape[0], 14), W3.dtype)], axis=1)
    xws3 = _tc_mid(p2, xws2, dis, b2.reshape(1, -1), W3p, 32, 16)
    p3 = _sc_agg(src3, dst3, xws3, 16)
    return _tc_last(p3, xws3, dis, b3.reshape(1, -1))


# speedup vs baseline: 47.1576x; 1.0495x over previous
import functools

import jax
import jax.numpy as jnp
from jax import lax
from jax.experimental import pallas as pl
from jax.experimental.pallas import tpu as pltpu
from jax.experimental.pallas import tpu_sc as plsc

N = 10000
NPAD = 10240
E = 320000
NC = 2
NS = 16
NW = NC * NS
EPW = E // NW
CE = 100
CH = EPW // CE
K = 10
SLICE = NPAD // NS

R = 2048
G = (N + R - 1) // R

_MESH = plsc.VectorSubcoreMesh(core_axis_name="c", subcore_axis_name="s")
_SC_PARAMS = pltpu.CompilerParams(use_tc_tiling_on_sc=False)


def _sc_degree(dst3):
    @functools.partial(
        pl.kernel,
        mesh=_MESH,
        out_type=jax.ShapeDtypeStruct((NC, NPAD), jnp.float32),
        compiler_params=_SC_PARAMS,
        scratch_types=[
            pltpu.VMEM((CH, CE), jnp.int32),
            pltpu.VMEM((112,), jnp.float32),
            pltpu.VMEM((SLICE,), jnp.float32),
            pltpu.VMEM_SHARED((NPAD,), jnp.float32),
        ],
    )
    def k(dst_hbm, out_hbm, dst_v, ones_v, zbuf_v, acc):
        c = lax.axis_index("c")
        s = lax.axis_index("s")
        wid = c * NS + s
        for i in range(112 // 16):
            ones_v[pl.ds(i * 16, 16)] = jnp.ones((16,), jnp.float32)
        for i in range(SLICE // 16):
            zbuf_v[pl.ds(i * 16, 16)] = jnp.zeros((16,), jnp.float32)
        pltpu.sync_copy(zbuf_v, acc.at[pl.ds(s * SLICE, SLICE)])
        plsc.subcore_barrier()
        pltpu.sync_copy(dst_hbm.at[wid], dst_v)

        def body(g, carry):
            pltpu.sync_copy(ones_v.at[pl.ds(0, CE)], acc.at[dst_v.at[g]],
                            add=True)
            return carry

        lax.fori_loop(0, CH, body, 0)
        plsc.subcore_barrier()
        pltpu.sync_copy(acc.at[pl.ds(s * SLICE, SLICE)],
                        out_hbm.at[c, pl.ds(s * SLICE, SLICE)])

    return k(dst3)


def _sc_agg(src3, dst3, xws, C):
    @functools.partial(
        pl.kernel,
        mesh=_MESH,
        out_type=jax.ShapeDtypeStruct((NC, NPAD, C), jnp.float32),
        compiler_params=_SC_PARAMS,
        scratch_types=[
            pltpu.VMEM((CH, CE), jnp.int32),
            pltpu.VMEM((CH, CE), jnp.int32),
            pltpu.VMEM((K, CE, C), jnp.float32),
            pltpu.VMEM_SHARED((NPAD, C), jnp.float32),
            pltpu.SemaphoreType.DMA((K,)),
            pltpu.SemaphoreType.DMA((K,)),
        ],
    )
    def k(src_hbm, dst_hbm, xws_hbm, z_hbm, out_hbm, src_v, dst_v, msg_v,
          acc, gsem, ssem):
        c = lax.axis_index("c")
        s = lax.axis_index("s")
        wid = c * NS + s
        pltpu.sync_copy(z_hbm, acc.at[pl.ds(s * SLICE, SLICE)])
        plsc.subcore_barrier()
        pltpu.sync_copy(src_hbm.at[wid], src_v)
        pltpu.sync_copy(dst_hbm.at[wid], dst_v)

        def body(i, carry):
            g0 = i * K
            for kk in range(K):
                @pl.when(i > 0)
                def _():
                    pltpu.make_async_copy(
                        msg_v.at[kk], acc.at[dst_v.at[g0 - K + kk]],
                        ssem.at[kk]).wait()
                pltpu.async_copy(xws_hbm.at[src_v.at[g0 + kk]],
                                 msg_v.at[kk], gsem.at[kk])
            for kk in range(K):
                pltpu.make_async_copy(xws_hbm.at[src_v.at[g0 + kk]],
                                      msg_v.at[kk], gsem.at[kk]).wait()
                pltpu.async_copy(msg_v.at[kk], acc.at[dst_v.at[g0 + kk]],
                                 ssem.at[kk], add=True)
            return carry

        lax.fori_loop(0, CH // K, body, 0)
        for kk in range(K):
            pltpu.make_async_copy(
                msg_v.at[kk], acc.at[dst_v.at[CH - K + kk]],
                ssem.at[kk]).wait()
        plsc.subcore_barrier()
        pltpu.sync_copy(acc.at[pl.ds(s * SLICE, SLICE)],
                        out_hbm.at[c, pl.ds(s * SLICE, SLICE)])

    return k(src3, dst3, xws, jnp.zeros((SLICE, C), jnp.float32))


def _tc1(x, degT, W1):
    def body(x_ref, degp_ref, w_ref, xws_ref, dis_ref):
        i = pl.program_id(0)
        dslc = degp_ref[:, pl.ds(i * R, R)]
        deg_row = dslc[0:1, :] + dslc[1:2, :] + 1.0
        dis = 1.0 / jnp.sqrt(jnp.transpose(deg_row))
        xw = jnp.dot(x_ref[...], w_ref[...], preferred_element_type=jnp.float32)
        xws_ref[...] = xw * dis
        dis_ref[...] = dis

    return pl.pallas_call(
        body,
        grid=(G,),
        in_specs=[
            pl.BlockSpec((R, 128), lambda i: (i, 0)),
            pl.BlockSpec((NC, NPAD), lambda i: (0, 0)),
            pl.BlockSpec((128, 64), lambda i: (0, 0)),
        ],
        out_specs=[
            pl.BlockSpec((R, 64), lambda i: (i, 0)),
            pl.BlockSpec((R, 1), lambda i: (i, 0)),
        ],
        out_shape=[
            jax.ShapeDtypeStruct((N, 64), jnp.float32),
            jax.ShapeDtypeStruct((N, 1), jnp.float32),
        ],
    )(x, degT, W1)


def _tc_mid(parts, xws, dis, b, W, C_in, C_out):
    def body(p_ref, xws_ref, dis_ref, b_ref, w_ref, out_ref):
        agg = p_ref[0] + p_ref[1] + xws_ref[...]
        h = jnp.maximum(dis_ref[...] * agg + b_ref[...], 0.0)
        hw = jnp.dot(h, w_ref[...], preferred_element_type=jnp.float32)
        out_ref[...] = hw * dis_ref[...]

    return pl.pallas_call(
        body,
        grid=(G,),
        in_specs=[
            pl.BlockSpec((NC, R, C_in), lambda i: (0, i, 0)),
            pl.BlockSpec((R, C_in), lambda i: (i, 0)),
            pl.BlockSpec((R, 1), lambda i: (i, 0)),
            pl.BlockSpec((1, C_in), lambda i: (0, 0)),
            pl.BlockSpec((C_in, C_out), lambda i: (0, 0)),
        ],
        out_specs=pl.BlockSpec((R, C_out), lambda i: (i, 0)),
        out_shape=jax.ShapeDtypeStruct((N, C_out), jnp.float32),
    )(parts, xws, dis, b, W)


def _tc_last(parts, xws, dis, b):
    def body(p_ref, xws_ref, dis_ref, b_ref, out_ref):
        agg = (p_ref[0] + p_ref[1] + xws_ref[...])[:, :2]
        out_ref[...] = jnp.maximum(dis_ref[...] * agg + b_ref[...], 0.0)

    return pl.pallas_call(
        body,
        grid=(G,),
        in_specs=[
            pl.BlockSpec((NC, R, 16), lambda i: (0, i, 0)),
            pl.BlockSpec((R, 16), lambda i: (i, 0)),
            pl.BlockSpec((R, 1), lambda i: (i, 0)),
            pl.BlockSpec((1, 2), lambda i: (0, 0)),
        ],
        out_specs=pl.BlockSpec((R, 2), lambda i: (i, 0)),
        out_shape=jax.ShapeDtypeStruct((N, 2), jnp.float32),
    )(parts, xws, dis, b)


def kernel(x, edge_index, W1, b1, W2, b2, W3, b3):
    src3 = edge_index[0].astype(jnp.int32).reshape(NW, CH, CE)
    dst3 = edge_index[1].astype(jnp.int32).reshape(NW, CH, CE)

    degp = _sc_degree(dst3)

    xws1, dis = _tc1(x, degp, W1)
    p1 = _sc_agg(src3, dst3, xws1, 64)
    xws2 = _tc_mid(p1, xws1, dis, b1.reshape(1, -1), W2, 64, 32)
    p2 = _sc_agg(src3, dst3, xws2, 32)
    W3p = jnp.concatenate([W3, jnp.zeros((W3.shape[0], 14), W3.dtype)], axis=1)
    xws3 = _tc_mid(p2, xws2, dis, b2.reshape(1, -1), W3p, 32, 16)
    p3 = _sc_agg(src3, dst3, xws3, 16)
    return _tc_last(p3, xws3, dis, b3.reshape(1, -1))


# speedup vs baseline: 47.7092x; 1.0117x over previous
import functools

import jax
import jax.numpy as jnp
from jax import lax
from jax.experimental import pallas as pl
from jax.experimental.pallas import tpu as pltpu
from jax.experimental.pallas import tpu_sc as plsc

N = 10000
NPAD = 10240
E = 320000
NC = 2
NS = 16
NW = NC * NS
EPW = E // NW
CE = 100
CH = EPW // CE
K = 10
SLICE = NPAD // NS

R = 2048
G = (N + R - 1) // R

_MESH = plsc.VectorSubcoreMesh(core_axis_name="c", subcore_axis_name="s")
_SC_PARAMS = pltpu.CompilerParams(use_tc_tiling_on_sc=False)


def _sc_degree(e4):
    @functools.partial(
        pl.kernel,
        mesh=_MESH,
        out_type=jax.ShapeDtypeStruct((NC, NPAD), jnp.float32),
        compiler_params=_SC_PARAMS,
        scratch_types=[
            pltpu.VMEM((CH, CE), jnp.int32),
            pltpu.VMEM((112,), jnp.float32),
            pltpu.VMEM((SLICE,), jnp.float32),
            pltpu.VMEM_SHARED((NPAD,), jnp.float32),
        ],
    )
    def k(dst_hbm, out_hbm, dst_v, ones_v, zbuf_v, acc):
        c = lax.axis_index("c")
        s = lax.axis_index("s")
        wid = c * NS + s
        for i in range(112 // 16):
            ones_v[pl.ds(i * 16, 16)] = jnp.ones((16,), jnp.float32)
        for i in range(SLICE // 16):
            zbuf_v[pl.ds(i * 16, 16)] = jnp.zeros((16,), jnp.float32)
        pltpu.sync_copy(zbuf_v, acc.at[pl.ds(s * SLICE, SLICE)])
        plsc.subcore_barrier()
        pltpu.sync_copy(dst_hbm.at[1, wid], dst_v)

        def body(g, carry):
            pltpu.sync_copy(ones_v.at[pl.ds(0, CE)], acc.at[dst_v.at[g]],
                            add=True)
            return carry

        lax.fori_loop(0, CH, body, 0)
        plsc.subcore_barrier()
        pltpu.sync_copy(acc.at[pl.ds(s * SLICE, SLICE)],
                        out_hbm.at[c, pl.ds(s * SLICE, SLICE)])

    return k(e4)


def _sc_agg(e4, xws, C):
    @functools.partial(
        pl.kernel,
        mesh=_MESH,
        out_type=jax.ShapeDtypeStruct((NC, NPAD, C), jnp.float32),
        compiler_params=_SC_PARAMS,
        scratch_types=[
            pltpu.VMEM((CH, CE), jnp.int32),
            pltpu.VMEM((CH, CE), jnp.int32),
            pltpu.VMEM((K, CE, C), jnp.float32),
            pltpu.VMEM_SHARED((NPAD, C), jnp.float32),
            pltpu.SemaphoreType.DMA((K,)),
            pltpu.SemaphoreType.DMA((K,)),
        ],
    )
    def k(e_hbm, xws_hbm, z_hbm, out_hbm, src_v, dst_v, msg_v,
          acc, gsem, ssem):
        c = lax.axis_index("c")
        s = lax.axis_index("s")
        wid = c * NS + s
        pltpu.sync_copy(z_hbm, acc.at[pl.ds(s * SLICE, SLICE)])
        plsc.subcore_barrier()
        pltpu.sync_copy(e_hbm.at[0, wid], src_v)
        pltpu.sync_copy(e_hbm.at[1, wid], dst_v)

        def body(i, carry):
            g0 = i * K
            for kk in range(K):
                @pl.when(i > 0)
                def _():
                    pltpu.make_async_copy(
                        msg_v.at[kk], acc.at[dst_v.at[g0 - K + kk]],
                        ssem.at[kk]).wait()
                pltpu.async_copy(xws_hbm.at[src_v.at[g0 + kk]],
                                 msg_v.at[kk], gsem.at[kk])
            for kk in range(K):
                pltpu.make_async_copy(xws_hbm.at[src_v.at[g0 + kk]],
                                      msg_v.at[kk], gsem.at[kk]).wait()
                pltpu.async_copy(msg_v.at[kk], acc.at[dst_v.at[g0 + kk]],
                                 ssem.at[kk], add=True)
            return carry

        lax.fori_loop(0, CH // K, body, 0)
        for kk in range(K):
            pltpu.make_async_copy(
                msg_v.at[kk], acc.at[dst_v.at[CH - K + kk]],
                ssem.at[kk]).wait()
        plsc.subcore_barrier()
        pltpu.sync_copy(acc.at[pl.ds(s * SLICE, SLICE)],
                        out_hbm.at[c, pl.ds(s * SLICE, SLICE)])

    return k(e4, xws, jnp.zeros((SLICE, C), jnp.float32))


def _tc1(x, degT, W1):
    def body(x_ref, degp_ref, w_ref, xws_ref, dis_ref):
        i = pl.program_id(0)
        dslc = degp_ref[:, pl.ds(i * R, R)]
        deg_row = dslc[0:1, :] + dslc[1:2, :] + 1.0
        dis = 1.0 / jnp.sqrt(jnp.transpose(deg_row))
        xw = jnp.dot(x_ref[...], w_ref[...], preferred_element_type=jnp.float32)
        xws_ref[...] = xw * dis
        dis_ref[...] = dis

    return pl.pallas_call(
        body,
        grid=(G,),
        in_specs=[
            pl.BlockSpec((R, 128), lambda i: (i, 0)),
            pl.BlockSpec((NC, NPAD), lambda i: (0, 0)),
            pl.BlockSpec((128, 64), lambda i: (0, 0)),
        ],
        out_specs=[
            pl.BlockSpec((R, 64), lambda i: (i, 0)),
            pl.BlockSpec((R, 1), lambda i: (i, 0)),
        ],
        out_shape=[
            jax.ShapeDtypeStruct((N, 64), jnp.float32),
            jax.ShapeDtypeStruct((N, 1), jnp.float32),
        ],
    )(x, degT, W1)


def _tc_mid(parts, xws, dis, b, W, C_in, C_out):
    def body(p_ref, xws_ref, dis_ref, b_ref, w_ref, out_ref):
        agg = p_ref[0] + p_ref[1] + xws_ref[...]
        h = jnp.maximum(dis_ref[...] * agg + b_ref[...], 0.0)
        hw = jnp.dot(h, w_ref[...], preferred_element_type=jnp.float32)
        out_ref[...] = hw * dis_ref[...]

    return pl.pallas_call(
        body,
        grid=(G,),
        in_specs=[
            pl.BlockSpec((NC, R, C_in), lambda i: (0, i, 0)),
            pl.BlockSpec((R, C_in), lambda i: (i, 0)),
            pl.BlockSpec((R, 1), lambda i: (i, 0)),
            pl.BlockSpec((1, C_in), lambda i: (0, 0)),
            pl.BlockSpec((C_in, C_out), lambda i: (0, 0)),
        ],
        out_specs=pl.BlockSpec((R, C_out), lambda i: (i, 0)),
        out_shape=jax.ShapeDtypeStruct((N, C_out), jnp.float32),
    )(parts, xws, dis, b, W)


def _tc_last(parts, xws, dis, b):
    def body(p_ref, xws_ref, dis_ref, b_ref, out_ref):
        agg = (p_ref[0] + p_ref[1] + xws_ref[...])[:, :2]
        out_ref[...] = jnp.maximum(dis_ref[...] * agg + b_ref[...], 0.0)

    return pl.pallas_call(
        body,
        grid=(G,),
        in_specs=[
            pl.BlockSpec((NC, R, 16), lambda i: (0, i, 0)),
            pl.BlockSpec((R, 16), lambda i: (i, 0)),
            pl.BlockSpec((R, 1), lambda i: (i, 0)),
            pl.BlockSpec((1, 2), lambda i: (0, 0)),
        ],
        out_specs=pl.BlockSpec((R, 2), lambda i: (i, 0)),
        out_shape=jax.ShapeDtypeStruct((N, 2), jnp.float32),
    )(parts, xws, dis, b)


def kernel(x, edge_index, W1, b1, W2, b2, W3, b3):
    e4 = edge_index.astype(jnp.int32).reshape(2, NW, CH, CE)

    degp = _sc_degree(e4)

    xws1, dis = _tc1(x, degp, W1)
    p1 = _sc_agg(e4, xws1, 64)
    xws2 = _tc_mid(p1, xws1, dis, b1.reshape(1, -1), W2, 64, 32)
    p2 = _sc_agg(e4, xws2, 32)
    W3p = jnp.concatenate([W3, jnp.zeros((W3.shape[0], 14), W3.dtype)], axis=1)
    xws3 = _tc_mid(p2, xws2, dis, b2.reshape(1, -1), W3p, 32, 16)
    p3 = _sc_agg(e4, xws3, 16)
    return _tc_last(p3, xws3, dis, b3.reshape(1, -1))


# speedup vs baseline: 49.5034x; 1.0376x over previous
import functools

import jax
import jax.numpy as jnp
from jax import lax
from jax.experimental import pallas as pl
from jax.experimental.pallas import tpu as pltpu
from jax.experimental.pallas import tpu_sc as plsc

N = 10000
NPAD = 10240
E = 320000
NC = 2
NS = 16
NW = NC * NS
EPW = E // NW
CE = 125
CH = EPW // CE
K = 8
SLICE = NPAD // NS

R = 2048
G = (N + R - 1) // R

_MESH = plsc.VectorSubcoreMesh(core_axis_name="c", subcore_axis_name="s")
_SC_PARAMS = pltpu.CompilerParams(use_tc_tiling_on_sc=False)


def _sc_degree(e4):
    @functools.partial(
        pl.kernel,
        mesh=_MESH,
        out_type=jax.ShapeDtypeStruct((NC, NPAD), jnp.float32),
        compiler_params=_SC_PARAMS,
        scratch_types=[
            pltpu.VMEM((CH, CE), jnp.int32),
            pltpu.VMEM((128,), jnp.float32),
            pltpu.VMEM((SLICE,), jnp.float32),
            pltpu.VMEM_SHARED((NPAD,), jnp.float32),
        ],
    )
    def k(dst_hbm, out_hbm, dst_v, ones_v, zbuf_v, acc):
        c = lax.axis_index("c")
        s = lax.axis_index("s")
        wid = c * NS + s
        for i in range(128 // 16):
            ones_v[pl.ds(i * 16, 16)] = jnp.ones((16,), jnp.float32)
        for i in range(SLICE // 16):
            zbuf_v[pl.ds(i * 16, 16)] = jnp.zeros((16,), jnp.float32)
        pltpu.sync_copy(zbuf_v, acc.at[pl.ds(s * SLICE, SLICE)])
        plsc.subcore_barrier()
        pltpu.sync_copy(dst_hbm.at[1, wid], dst_v)

        def body(g, carry):
            pltpu.sync_copy(ones_v.at[pl.ds(0, CE)], acc.at[dst_v.at[g]],
                            add=True)
            return carry

        lax.fori_loop(0, CH, body, 0)
        plsc.subcore_barrier()
        pltpu.sync_copy(acc.at[pl.ds(s * SLICE, SLICE)],
                        out_hbm.at[c, pl.ds(s * SLICE, SLICE)])

    return k(e4)


def _sc_agg(e4, xws, C):
    @functools.partial(
        pl.kernel,
        mesh=_MESH,
        out_type=jax.ShapeDtypeStruct((NC, NPAD, C), jnp.float32),
        compiler_params=_SC_PARAMS,
        scratch_types=[
            pltpu.VMEM((CH, CE), jnp.int32),
            pltpu.VMEM((CH, CE), jnp.int32),
            pltpu.VMEM((K, CE, C), jnp.float32),
            pltpu.VMEM_SHARED((NPAD, C), jnp.float32),
            pltpu.SemaphoreType.DMA((K,)),
            pltpu.SemaphoreType.DMA((K,)),
        ],
    )
    def k(e_hbm, xws_hbm, z_hbm, out_hbm, src_v, dst_v, msg_v,
          acc, gsem, ssem):
        c = lax.axis_index("c")
        s = lax.axis_index("s")
        wid = c * NS + s
        pltpu.sync_copy(z_hbm, acc.at[pl.ds(s * SLICE, SLICE)])
        plsc.subcore_barrier()
        pltpu.sync_copy(e_hbm.at[0, wid], src_v)
        pltpu.sync_copy(e_hbm.at[1, wid], dst_v)

        def body(i, carry):
            g0 = i * K
            for kk in range(K):
                @pl.when(i > 0)
                def _():
                    pltpu.make_async_copy(
                        msg_v.at[kk], acc.at[dst_v.at[g0 - K + kk]],
                        ssem.at[kk]).wait()
                pltpu.async_copy(xws_hbm.at[src_v.at[g0 + kk]],
                                 msg_v.at[kk], gsem.at[kk])
            for kk in range(K):
                pltpu.make_async_copy(xws_hbm.at[src_v.at[g0 + kk]],
                                      msg_v.at[kk], gsem.at[kk]).wait()
                pltpu.async_copy(msg_v.at[kk], acc.at[dst_v.at[g0 + kk]],
                                 ssem.at[kk], add=True)
            return carry

        lax.fori_loop(0, CH // K, body, 0)
        for kk in range(K):
            pltpu.make_async_copy(
                msg_v.at[kk], acc.at[dst_v.at[CH - K + kk]],
                ssem.at[kk]).wait()
        plsc.subcore_barrier()
        pltpu.sync_copy(acc.at[pl.ds(s * SLICE, SLICE)],
                        out_hbm.at[c, pl.ds(s * SLICE, SLICE)])

    return k(e4, xws, jnp.zeros((SLICE, C), jnp.float32))


def _tc1(x, degT, W1):
    def body(x_ref, degp_ref, w_ref, xws_ref, dis_ref):
        i = pl.program_id(0)
        dslc = degp_ref[:, pl.ds(i * R, R)]
        deg_row = dslc[0:1, :] + dslc[1:2, :] + 1.0
        dis = 1.0 / jnp.sqrt(jnp.transpose(deg_row))
        xw = jnp.dot(x_ref[...], w_ref[...], preferred_element_type=jnp.float32)
        xws_ref[...] = xw * dis
        dis_ref[...] = dis

    return pl.pallas_call(
        body,
        grid=(G,),
        in_specs=[
            pl.BlockSpec((R, 128), lambda i: (i, 0)),
            pl.BlockSpec((NC, NPAD), lambda i: (0, 0)),
            pl.BlockSpec((128, 64), lambda i: (0, 0)),
        ],
        out_specs=[
            pl.BlockSpec((R, 64), lambda i: (i, 0)),
            pl.BlockSpec((R, 1), lambda i: (i, 0)),
        ],
        out_shape=[
            jax.ShapeDtypeStruct((N, 64), jnp.float32),
            jax.ShapeDtypeStruct((N, 1), jnp.float32),
        ],
    )(x, degT, W1)


def _tc_mid(parts, xws, dis, b, W, C_in, C_out):
    def body(p_ref, xws_ref, dis_ref, b_ref, w_ref, out_ref):
        agg = p_ref[0] + p_ref[1] + xws_ref[...]
        h = jnp.maximum(dis_ref[...] * agg + b_ref[...], 0.0)
        hw = jnp.dot(h, w_ref[...], preferred_element_type=jnp.float32)
        out_ref[...] = hw * dis_ref[...]

    return pl.pallas_call(
        body,
        grid=(G,),
        in_specs=[
            pl.BlockSpec((NC, R, C_in), lambda i: (0, i, 0)),
            pl.BlockSpec((R, C_in), lambda i: (i, 0)),
            pl.BlockSpec((R, 1), lambda i: (i, 0)),
            pl.BlockSpec((1, C_in), lambda i: (0, 0)),
            pl.BlockSpec((C_in, C_out), lambda i: (0, 0)),
        ],
        out_specs=pl.BlockSpec((R, C_out), lambda i: (i, 0)),
        out_shape=jax.ShapeDtypeStruct((N, C_out), jnp.float32),
    )(parts, xws, dis, b, W)


def _tc_last(parts, xws, dis, b):
    def body(p_ref, xws_ref, dis_ref, b_ref, out_ref):
        agg = (p_ref[0] + p_ref[1] + xws_ref[...])[:, :2]
        out_ref[...] = jnp.maximum(dis_ref[...] * agg + b_ref[...], 0.0)

    return pl.pallas_call(
        body,
        grid=(G,),
        in_specs=[
            pl.BlockSpec((NC, R, 16), lambda i: (0, i, 0)),
            pl.BlockSpec((R, 16), lambda i: (i, 0)),
            pl.BlockSpec((R, 1), lambda i: (i, 0)),
            pl.BlockSpec((1, 2), lambda i: (0, 0)),
        ],
        out_specs=pl.BlockSpec((R, 2), lambda i: (i, 0)),
        out_shape=jax.ShapeDtypeStruct((N, 2), jnp.float32),
    )(parts, xws, dis, b)


def kernel(x, edge_index, W1, b1, W2, b2, W3, b3):
    e4 = edge_index.astype(jnp.int32).reshape(2, NW, CH, CE)

    degp = _sc_degree(e4)

    xws1, dis = _tc1(x, degp, W1)
    p1 = _sc_agg(e4, xws1, 64)
    xws2 = _tc_mid(p1, xws1, dis, b1.reshape(1, -1), W2, 64, 32)
    p2 = _sc_agg(e4, xws2, 32)
    W3p = jnp.concatenate([W3, jnp.zeros((W3.shape[0], 14), W3.dtype)], axis=1)
    xws3 = _tc_mid(p2, xws2, dis, b2.reshape(1, -1), W3p, 32, 16)
    p3 = _sc_agg(e4, xws3, 16)
    return _tc_last(p3, xws3, dis, b3.reshape(1, -1))
